# Initial kernel scaffold; baseline (speedup 1.0000x reference)
import jax
import jax.numpy as jnp
from jax.experimental import pallas as pl


def kernel(x, in_proj_w, conv_w, conv_b, x_proj_w, dt_proj_w, dt_proj_b, A_log, D, out_proj_w):
    raise NotImplementedError("write your pallas kernel here")


# speedup vs baseline: 11.1741x; 11.1741x over previous
import functools

import jax
import jax.numpy as jnp
from jax.experimental import pallas as pl
from jax.experimental.pallas import tpu as pltpu

D_MODEL = 280
D_STATE = 16
D_CONV = 4
N_LAYERS = 3
D_INNER = 560
DT_RANK = 18

TA = 512
TB = 128


def _dot(a, b):
    return jnp.dot(a.astype(jnp.bfloat16), b.astype(jnp.bfloat16),
                   preferred_element_type=jnp.float32)


def _proj_kernel(x_ref, inw_ref, cw_ref, cb_ref, xpw_ref, dtw_ref, dtb_ref,
                 xc_ref, dt_ref, z_ref, bc_ref, xp_scr):
    j = pl.program_id(1)
    x = x_ref[0]
    xz = _dot(x, inw_ref[...].T)
    xin = xz[:, :D_INNER]
    z_ref[0] = xz[:, D_INNER:]

    @pl.when(j == 0)
    def _():
        xp_scr[0:8, :] = jnp.zeros((8, D_INNER), jnp.float32)

    @pl.when(j > 0)
    def _():
        xp_scr[0:8, :] = xp_scr[TA:TA + 8, :]

    xp_scr[8:TA + 8, :] = xin
    acc = cb_ref[...]
    conv = (xp_scr[5:5 + TA, :] * cw_ref[:, 0][None, :]
            + xp_scr[6:6 + TA, :] * cw_ref[:, 1][None, :]
            + xp_scr[7:7 + TA, :] * cw_ref[:, 2][None, :]
            + xp_scr[8:8 + TA, :] * cw_ref[:, 3][None, :]
            + acc)
    xc = conv * jax.nn.sigmoid(conv)
    xc_ref[0] = xc

    proj = _dot(xc, xpw_ref[...].T)
    bc_ref[0] = proj[:, DT_RANK:DT_RANK + 2 * D_STATE]
    dt_raw = proj[:, :DT_RANK]
    dtl = _dot(dt_raw, dtw_ref[...].T) + dtb_ref[...]
    dt_ref[0] = jax.nn.softplus(dtl)


def _layer_proj(x, inw, cw, cb, xpw, dtw, dtb):
    B, L, _ = x.shape
    grid = (B, L // TA)
    out_shapes = [
        jax.ShapeDtypeStruct((B, L, D_INNER), jnp.float32),
        jax.ShapeDtypeStruct((B, L, D_INNER), jnp.float32),
        jax.ShapeDtypeStruct((B, L, D_INNER), jnp.float32),
        jax.ShapeDtypeStruct((B, L, 2 * D_STATE), jnp.float32),
    ]
    full = lambda shape: pl.BlockSpec(shape, lambda b, j: (0,) * len(shape))
    return pl.pallas_call(
        _proj_kernel,
        grid=grid,
        in_specs=[
            pl.BlockSpec((1, TA, D_MODEL), lambda b, j: (b, j, 0)),
            full((2 * D_INNER, D_MODEL)),
            full((D_INNER, D_CONV)),
            full((1, D_INNER)),
            full((DT_RANK + 2 * D_STATE, D_INNER)),
            full((D_INNER, DT_RANK)),
            full((1, D_INNER)),
        ],
        out_specs=[
            pl.BlockSpec((1, TA, D_INNER), lambda b, j: (b, j, 0)),
            pl.BlockSpec((1, TA, D_INNER), lambda b, j: (b, j, 0)),
            pl.BlockSpec((1, TA, D_INNER), lambda b, j: (b, j, 0)),
            pl.BlockSpec((1, TA, 2 * D_STATE), lambda b, j: (b, j, 0)),
        ],
        out_shape=out_shapes,
        scratch_shapes=[pltpu.VMEM((TA + 8, D_INNER), jnp.float32)],
        compiler_params=pltpu.CompilerParams(
            dimension_semantics=("parallel", "arbitrary"),
        ),
        name="mamba_proj",
    )(x, inw, cw, cb.reshape(1, D_INNER), xpw, dtw, dtb.reshape(1, D_INNER))


def _scan_kernel(xc_ref, dt_ref, z_ref, bc_ref, alt_ref, d_ref, owt_ref,
                 out_ref, h_scr, p_scr, q_scr, hist_scr):
    j = pl.program_id(1)

    @pl.when(j == 0)
    def _():
        h_scr[...] = jnp.zeros((D_STATE, D_INNER), jnp.float32)

    A = -jnp.exp(alt_ref[...])
    dt = dt_ref[0]
    xc = xc_ref[0]
    bc = bc_ref[0]
    p_scr[...] = jnp.exp(dt[:, None, :] * A[None])
    dtu = dt * xc
    q_scr[...] = dtu[:, None, :] * bc[:, :D_STATE][:, :, None]

    def body(s, carry):
        h = p_scr[s] * h_scr[...] + q_scr[s]
        h_scr[...] = h
        hist_scr[s] = h
        return carry

    jax.lax.fori_loop(0, TB, body, 0)

    cm = bc[:, D_STATE:2 * D_STATE][:, :, None]
    y = jnp.sum(cm * hist_scr[...], axis=1)
    y = y + xc * d_ref[...]
    z = z_ref[0]
    yg = y * (z * jax.nn.sigmoid(z))
    out_ref[0] = _dot(yg, owt_ref[...])


def _layer_scan(xc, dt, z, bc, A_log, Dv, ow):
    B, L, _ = xc.shape
    grid = (B, L // TB)
    full = lambda shape: pl.BlockSpec(shape, lambda b, j: (0,) * len(shape))
    return pl.pallas_call(
        _scan_kernel,
        grid=grid,
        in_specs=[
            pl.BlockSpec((1, TB, D_INNER), lambda b, j: (b, j, 0)),
            pl.BlockSpec((1, TB, D_INNER), lambda b, j: (b, j, 0)),
            pl.BlockSpec((1, TB, D_INNER), lambda b, j: (b, j, 0)),
            pl.BlockSpec((1, TB, 2 * D_STATE), lambda b, j: (b, j, 0)),
            full((D_STATE, D_INNER)),
            full((1, D_INNER)),
            full((D_INNER, D_MODEL)),
        ],
        out_specs=pl.BlockSpec((1, TB, D_MODEL), lambda b, j: (b, j, 0)),
        out_shape=jax.ShapeDtypeStruct((B, L, D_MODEL), jnp.float32),
        scratch_shapes=[
            pltpu.VMEM((D_STATE, D_INNER), jnp.float32),
            pltpu.VMEM((TB, D_STATE, D_INNER), jnp.float32),
            pltpu.VMEM((TB, D_STATE, D_INNER), jnp.float32),
            pltpu.VMEM((TB, D_STATE, D_INNER), jnp.float32),
        ],
        compiler_params=pltpu.CompilerParams(
            dimension_semantics=("parallel", "arbitrary"),
        ),
        name="mamba_scan",
    )(xc, dt, z, bc, A_log.T, Dv.reshape(1, D_INNER), ow.T)


@jax.jit
def kernel(x, in_proj_w, conv_w, conv_b, x_proj_w, dt_proj_w, dt_proj_b,
           A_log, D, out_proj_w):
    for i in range(N_LAYERS):
        xc, dt, z, bc = _layer_proj(x, in_proj_w[i], conv_w[i], conv_b[i],
                                    x_proj_w[i], dt_proj_w[i], dt_proj_b[i])
        x = _layer_scan(xc, dt, z, bc, A_log[i], D[i], out_proj_w[i])
    return x


# speedup vs baseline: 12.6897x; 1.1356x over previous
import functools

import jax
import jax.numpy as jnp
from jax.experimental import pallas as pl
from jax.experimental.pallas import tpu as pltpu

D_MODEL = 280
D_STATE = 16
D_CONV = 4
N_LAYERS = 3
D_INNER = 560
DT_RANK = 18

TA = 512
TB = 128


def _dot(a, b):
    return jnp.dot(a.astype(jnp.bfloat16), b.astype(jnp.bfloat16),
                   preferred_element_type=jnp.float32)


def _proj_kernel(x_ref, inw_ref, cw_ref, cb_ref, xpw_ref, dtw_ref, dtb_ref,
                 xc_ref, dt_ref, z_ref, bc_ref, xp_scr):
    j = pl.program_id(1)
    x = x_ref[0]
    xz = _dot(x, inw_ref[...])
    xin = xz[:, :D_INNER]
    z_ref[0] = xz[:, D_INNER:]

    @pl.when(j == 0)
    def _():
        xp_scr[0:8, :] = jnp.zeros((8, D_INNER), jnp.float32)

    @pl.when(j > 0)
    def _():
        xp_scr[0:8, :] = xp_scr[TA:TA + 8, :]

    xp_scr[8:TA + 8, :] = xin
    acc = cb_ref[...]
    conv = (xp_scr[5:5 + TA, :] * cw_ref[:, 0][None, :]
            + xp_scr[6:6 + TA, :] * cw_ref[:, 1][None, :]
            + xp_scr[7:7 + TA, :] * cw_ref[:, 2][None, :]
            + xp_scr[8:8 + TA, :] * cw_ref[:, 3][None, :]
            + acc)
    xc = conv * jax.nn.sigmoid(conv)
    xc_ref[0] = xc

    proj = _dot(xc, xpw_ref[...])
    bc_ref[0] = proj[:, DT_RANK:DT_RANK + 2 * D_STATE]
    dt_raw = proj[:, :DT_RANK]
    dtl = _dot(dt_raw, dtw_ref[...]) + dtb_ref[...]
    dt_ref[0] = jax.nn.softplus(dtl)


def _layer_proj(x, inw, cw, cb, xpw, dtw, dtb):
    B, L, _ = x.shape
    grid = (B, L // TA)
    out_shapes = [
        jax.ShapeDtypeStruct((B, L, D_INNER), jnp.float32),
        jax.ShapeDtypeStruct((B, L, D_INNER), jnp.float32),
        jax.ShapeDtypeStruct((B, L, D_INNER), jnp.float32),
        jax.ShapeDtypeStruct((B, L, 2 * D_STATE), jnp.float32),
    ]
    full = lambda shape: pl.BlockSpec(shape, lambda b, j: (0,) * len(shape))
    return pl.pallas_call(
        _proj_kernel,
        grid=grid,
        in_specs=[
            pl.BlockSpec((1, TA, D_MODEL), lambda b, j: (b, j, 0)),
            full((D_MODEL, 2 * D_INNER)),
            full((D_INNER, D_CONV)),
            full((1, D_INNER)),
            full((D_INNER, DT_RANK + 2 * D_STATE)),
            full((DT_RANK, D_INNER)),
            full((1, D_INNER)),
        ],
        out_specs=[
            pl.BlockSpec((1, TA, D_INNER), lambda b, j: (b, j, 0)),
            pl.BlockSpec((1, TA, D_INNER), lambda b, j: (b, j, 0)),
            pl.BlockSpec((1, TA, D_INNER), lambda b, j: (b, j, 0)),
            pl.BlockSpec((1, TA, 2 * D_STATE), lambda b, j: (b, j, 0)),
        ],
        out_shape=out_shapes,
        scratch_shapes=[pltpu.VMEM((TA + 8, D_INNER), jnp.float32)],
        compiler_params=pltpu.CompilerParams(
            dimension_semantics=("parallel", "arbitrary"),
        ),
        name="mamba_proj",
    )(x, inw.T, cw, cb.reshape(1, D_INNER), xpw.T, dtw.T,
      dtb.reshape(1, D_INNER))


def _scan_kernel(xc_ref, dt_ref, z_ref, bc_ref, alt_ref, d_ref, owt_ref,
                 out_ref, h_scr, p_scr, q_scr, hist_scr):
    j = pl.program_id(1)

    @pl.when(j == 0)
    def _():
        h_scr[...] = jnp.zeros((D_STATE, D_INNER), jnp.float32)

    A = -jnp.exp(alt_ref[...])
    dt = dt_ref[0]
    xc = xc_ref[0]
    bc = bc_ref[0]
    p_scr[...] = jnp.exp(dt[:, None, :] * A[None])
    dtu = dt * xc
    q_scr[...] = dtu[:, None, :] * bc[:, :D_STATE][:, :, None]

    def body(s, carry):
        h = p_scr[s] * h_scr[...] + q_scr[s]
        h_scr[...] = h
        hist_scr[s] = h
        return carry

    jax.lax.fori_loop(0, TB, body, 0)

    tbn = TB * D_STATE
    cm = bc[:, D_STATE:2 * D_STATE]
    n_io = jax.lax.broadcasted_iota(jnp.int32, (D_STATE, tbn), 0)
    k_io0 = jax.lax.broadcasted_iota(jnp.int32, (D_STATE, tbn), 1)
    rep = ((k_io0 & (D_STATE - 1)) == n_io).astype(jnp.bfloat16)
    cmk = jnp.dot(cm.astype(jnp.bfloat16), rep,
                  preferred_element_type=jnp.float32)
    t_io = jax.lax.broadcasted_iota(jnp.int32, (TB, tbn), 0)
    k_io = jax.lax.broadcasted_iota(jnp.int32, (TB, tbn), 1)
    smat = jnp.where((k_io >> 4) == t_io, cmk, 0.0)
    hf = hist_scr[...].reshape(tbn, D_INNER)
    y = jnp.dot(smat.astype(jnp.bfloat16), hf.astype(jnp.bfloat16),
                preferred_element_type=jnp.float32)
    y = y + xc * d_ref[...]
    z = z_ref[0]
    yg = y * (z * jax.nn.sigmoid(z))
    out_ref[0] = _dot(yg, owt_ref[...])


def _layer_scan(xc, dt, z, bc, A_log, Dv, ow):
    B, L, _ = xc.shape
    grid = (B, L // TB)
    full = lambda shape: pl.BlockSpec(shape, lambda b, j: (0,) * len(shape))
    return pl.pallas_call(
        _scan_kernel,
        grid=grid,
        in_specs=[
            pl.BlockSpec((1, TB, D_INNER), lambda b, j: (b, j, 0)),
            pl.BlockSpec((1, TB, D_INNER), lambda b, j: (b, j, 0)),
            pl.BlockSpec((1, TB, D_INNER), lambda b, j: (b, j, 0)),
            pl.BlockSpec((1, TB, 2 * D_STATE), lambda b, j: (b, j, 0)),
            full((D_STATE, D_INNER)),
            full((1, D_INNER)),
            full((D_INNER, D_MODEL)),
        ],
        out_specs=pl.BlockSpec((1, TB, D_MODEL), lambda b, j: (b, j, 0)),
        out_shape=jax.ShapeDtypeStruct((B, L, D_MODEL), jnp.float32),
        scratch_shapes=[
            pltpu.VMEM((D_STATE, D_INNER), jnp.float32),
            pltpu.VMEM((TB, D_STATE, D_INNER), jnp.float32),
            pltpu.VMEM((TB, D_STATE, D_INNER), jnp.float32),
            pltpu.VMEM((TB, D_STATE, D_INNER), jnp.float32),
        ],
        compiler_params=pltpu.CompilerParams(
            dimension_semantics=("parallel", "arbitrary"),
        ),
        name="mamba_scan",
    )(xc, dt, z, bc, A_log.T, Dv.reshape(1, D_INNER), ow.T)


@jax.jit
def kernel(x, in_proj_w, conv_w, conv_b, x_proj_w, dt_proj_w, dt_proj_b,
           A_log, D, out_proj_w):
    for i in range(N_LAYERS):
        xc, dt, z, bc = _layer_proj(x, in_proj_w[i], conv_w[i], conv_b[i],
                                    x_proj_w[i], dt_proj_w[i], dt_proj_b[i])
        x = _layer_scan(xc, dt, z, bc, A_log[i], D[i], out_proj_w[i])
    return x
